# baseline (speedup 1.0000x reference)
import functools

import jax
import jax.numpy as jnp
from jax.experimental import pallas as pl
from jax.experimental.pallas import tpu as pltpu

_CBLK = 4000


def _matmul_block(x_ref, m_ref, o_ref):
    o_ref[...] = jax.lax.dot_general(
        m_ref[...].astype(jnp.bfloat16),
        x_ref[...].astype(jnp.bfloat16),
        dimension_numbers=(((1,), (1,)), ((), ())),
        preferred_element_type=jnp.float32,
    )


@functools.partial(jax.jit, static_argnames=())
def kernel(inputs, targets, mem):
    del targets
    b, f = inputs.shape
    c = mem.shape[0]
    grid = (pl.cdiv(c, _CBLK),)
    out_t = pl.pallas_call(
        _matmul_block,
        grid=grid,
        in_specs=[
            pl.BlockSpec((b, f), lambda i: (0, 0)),
            pl.BlockSpec((_CBLK, f), lambda i: (i, 0)),
        ],
        out_specs=pl.BlockSpec((_CBLK, b), lambda i: (i, 0)),
        out_shape=jax.ShapeDtypeStruct((c, b), jnp.float32),
        compiler_params=pltpu.CompilerParams(
            dimension_semantics=("arbitrary",),
        ),
    )(inputs, mem)
    return out_t.T
